# initial kernel scaffold (unmeasured)
import jax
import jax.numpy as jnp
from jax import lax
from jax.experimental import pallas as pl
from jax.experimental.pallas import tpu as pltpu

N_DEV = 8
SQ = 1024
SKV = 1024
D_MODEL = 1024
H_PER = 8
DH = 128
CHUNK = SQ // N_DEV
SCALE = 0.08838834764831843


def kernel(x, Wq, K_ext, V_ext, Wo):
    def body(x_ref, wq_ref, k_hbm, v_hbm, wo_ref, out_ref,
             k_vmem, v_vmem, acc_ref, rs_recv,
             local_sems, rs_send_sems, rs_recv_sems,
             ag_send_sems, ag_recv_sems):
        my = lax.axis_index("i")
        right = lax.rem(my + 1, N_DEV)
        left = lax.rem(my + N_DEV - 1, N_DEV)

        barrier = pltpu.get_barrier_semaphore()
        pl.semaphore_signal(barrier, inc=1, device_id=(left,),
                            device_id_type=pl.DeviceIdType.MESH)
        pl.semaphore_signal(barrier, inc=1, device_id=(right,),
                            device_id_type=pl.DeviceIdType.MESH)
        pl.semaphore_wait(barrier, 2)

        kcopy = pltpu.make_async_copy(
            k_hbm.at[0, :, pl.ds(my * H_PER, H_PER), :], k_vmem,
            local_sems.at[0])
        vcopy = pltpu.make_async_copy(
            v_hbm.at[0, :, pl.ds(my * H_PER, H_PER), :], v_vmem,
            local_sems.at[1])
        kcopy.start()
        vcopy.start()

        xb = x_ref[0].astype(jnp.bfloat16)
        wqb = wq_ref[...].astype(jnp.bfloat16)
        q = jnp.dot(xb, wqb, preferred_element_type=jnp.float32)

        qi = lax.broadcasted_iota(jnp.int32, (SQ, SKV), 0)
        ki = lax.broadcasted_iota(jnp.int32, (SQ, SKV), 1)
        mask = (jnp.abs(qi - ki) <= 128) | (ki < 32) | (qi < 32)

        kcopy.wait()
        vcopy.wait()

        acc = jnp.zeros((SQ, D_MODEL), jnp.float32)
        for h in range(H_PER):
            qh = q[:, h * DH:(h + 1) * DH].astype(jnp.bfloat16)
            kh = k_vmem[:, h, :].astype(jnp.bfloat16)
            s = lax.dot_general(qh, kh, (((1,), (1,)), ((), ())),
                                preferred_element_type=jnp.float32)
            s = jnp.where(mask, s * SCALE, -1e9)
            m = jnp.max(s, axis=1, keepdims=True)
            w = jnp.exp(s - m)
            w = w / jnp.sum(w, axis=1, keepdims=True)
            vh = v_vmem[:, h, :].astype(jnp.bfloat16)
            ctx = jnp.dot(w.astype(jnp.bfloat16), vh,
                          preferred_element_type=jnp.float32)
            wo_h = wo_ref[h * DH:(h + 1) * DH, :].astype(jnp.bfloat16)
            acc = acc + jnp.dot(ctx.astype(jnp.bfloat16), wo_h,
                                preferred_element_type=jnp.float32)
        acc_ref[...] = acc

        for s in range(N_DEV - 1):
            send_idx = lax.rem(my + 2 * N_DEV - s, N_DEV)
            recv_idx = lax.rem(my + 2 * N_DEV - 1 - s, N_DEV)
            rdma = pltpu.make_async_remote_copy(
                src_ref=acc_ref.at[pl.ds(send_idx * CHUNK, CHUNK), :],
                dst_ref=rs_recv.at[s],
                send_sem=rs_send_sems.at[s],
                recv_sem=rs_recv_sems.at[s],
                device_id=(right,),
                device_id_type=pl.DeviceIdType.MESH,
            )
            rdma.start()
            rdma.wait()
            cur = acc_ref[pl.ds(recv_idx * CHUNK, CHUNK), :]
            acc_ref[pl.ds(recv_idx * CHUNK, CHUNK), :] = cur + rs_recv[s]

        c = lax.rem(my + 1, N_DEV)
        out_ref[0, pl.ds(c * CHUNK, CHUNK), :] = \
            acc_ref[pl.ds(c * CHUNK, CHUNK), :]

        for s in range(N_DEV - 1):
            send_idx = lax.rem(my + 1 + 2 * N_DEV - s, N_DEV)
            rdma = pltpu.make_async_remote_copy(
                src_ref=out_ref.at[0, pl.ds(send_idx * CHUNK, CHUNK), :],
                dst_ref=out_ref.at[0, pl.ds(send_idx * CHUNK, CHUNK), :],
                send_sem=ag_send_sems.at[s],
                recv_sem=ag_recv_sems.at[s],
                device_id=(right,),
                device_id_type=pl.DeviceIdType.MESH,
            )
            rdma.start()
            rdma.wait()

    return pl.pallas_call(
        body,
        out_shape=jax.ShapeDtypeStruct((1, SQ, D_MODEL), jnp.float32),
        in_specs=[
            pl.BlockSpec(memory_space=pltpu.MemorySpace.VMEM),
            pl.BlockSpec(memory_space=pltpu.MemorySpace.VMEM),
            pl.BlockSpec(memory_space=pltpu.MemorySpace.ANY),
            pl.BlockSpec(memory_space=pltpu.MemorySpace.ANY),
            pl.BlockSpec(memory_space=pltpu.MemorySpace.VMEM),
        ],
        out_specs=pl.BlockSpec(memory_space=pltpu.MemorySpace.VMEM),
        scratch_shapes=[
            pltpu.VMEM((SKV, H_PER, DH), jnp.float32),
            pltpu.VMEM((SKV, H_PER, DH), jnp.float32),
            pltpu.VMEM((SQ, D_MODEL), jnp.float32),
            pltpu.VMEM((N_DEV - 1, CHUNK, D_MODEL), jnp.float32),
            pltpu.SemaphoreType.DMA((2,)),
            pltpu.SemaphoreType.DMA((N_DEV - 1,)),
            pltpu.SemaphoreType.DMA((N_DEV - 1,)),
            pltpu.SemaphoreType.DMA((N_DEV - 1,)),
            pltpu.SemaphoreType.DMA((N_DEV - 1,)),
        ],
        compiler_params=pltpu.CompilerParams(collective_id=0),
    )(x, Wq, K_ext, V_ext, Wo)


# baseline (device time: 148872 ns/iter reference)
import jax
import jax.numpy as jnp
from jax import lax
from jax.experimental import pallas as pl
from jax.experimental.pallas import tpu as pltpu

N_DEV = 8
SQ = 1024
SKV = 1024
D_MODEL = 1024
H_PER = 8
DH = 128
CHUNK = SQ // N_DEV
SCALE = 0.08838834764831843


def kernel(x, Wq, K_ext, V_ext, Wo):
    def body(x_ref, wq_ref, k_hbm, v_hbm, wo_ref, out_ref,
             k_vmem, v_vmem, acc_ref, rs_recv,
             local_sems, rs_send_sems, rs_recv_sems,
             ag_send_sems, ag_recv_sems):
        my = lax.axis_index("i")
        right = lax.rem(my + 1, N_DEV)
        left = lax.rem(my + N_DEV - 1, N_DEV)

        barrier = pltpu.get_barrier_semaphore()
        pl.semaphore_signal(barrier, inc=1, device_id=(left,),
                            device_id_type=pl.DeviceIdType.MESH)
        pl.semaphore_signal(barrier, inc=1, device_id=(right,),
                            device_id_type=pl.DeviceIdType.MESH)
        pl.semaphore_wait(barrier, 2)

        kcopy = pltpu.make_async_copy(
            k_hbm.at[0, :, pl.ds(my * H_PER, H_PER), :], k_vmem,
            local_sems.at[0])
        vcopy = pltpu.make_async_copy(
            v_hbm.at[0, :, pl.ds(my * H_PER, H_PER), :], v_vmem,
            local_sems.at[1])
        kcopy.start()
        vcopy.start()

        xb = x_ref[0].astype(jnp.bfloat16)
        wqb = wq_ref[...].astype(jnp.bfloat16)
        q = jnp.dot(xb, wqb, preferred_element_type=jnp.float32)

        qi = lax.broadcasted_iota(jnp.int32, (SQ, SKV), 0)
        ki = lax.broadcasted_iota(jnp.int32, (SQ, SKV), 1)
        mask = (jnp.abs(qi - ki) <= 128) | (ki < 32) | (qi < 32)

        kcopy.wait()
        vcopy.wait()

        acc = jnp.zeros((SQ, D_MODEL), jnp.float32)
        for h in range(H_PER):
            qh = q[:, h * DH:(h + 1) * DH].astype(jnp.bfloat16)
            kh = k_vmem[:, h, :].astype(jnp.bfloat16)
            s = lax.dot_general(qh, kh, (((1,), (1,)), ((), ())),
                                preferred_element_type=jnp.float32)
            s = jnp.where(mask, s * SCALE, -1e9)
            m = jnp.max(s, axis=1, keepdims=True)
            w = jnp.exp(s - m)
            w = w / jnp.sum(w, axis=1, keepdims=True)
            vh = v_vmem[:, h, :].astype(jnp.bfloat16)
            ctx = jnp.dot(w.astype(jnp.bfloat16), vh,
                          preferred_element_type=jnp.float32)
            wo_h = wo_ref[h * DH:(h + 1) * DH, :].astype(jnp.bfloat16)
            acc = acc + jnp.dot(ctx.astype(jnp.bfloat16), wo_h,
                                preferred_element_type=jnp.float32)
        acc_ref[...] = acc

        for s in range(N_DEV - 1):
            send_idx = lax.rem(my + 2 * N_DEV - s, N_DEV)
            recv_idx = lax.rem(my + 2 * N_DEV - 1 - s, N_DEV)
            rdma = pltpu.make_async_remote_copy(
                src_ref=acc_ref.at[pl.ds(send_idx * CHUNK, CHUNK), :],
                dst_ref=rs_recv.at[s],
                send_sem=rs_send_sems.at[s],
                recv_sem=rs_recv_sems.at[s],
                device_id=(right,),
                device_id_type=pl.DeviceIdType.MESH,
            )
            rdma.start()
            rdma.wait()
            cur = acc_ref[pl.ds(recv_idx * CHUNK, CHUNK), :]
            acc_ref[pl.ds(recv_idx * CHUNK, CHUNK), :] = cur + rs_recv[s]

        c = lax.rem(my + 1, N_DEV)
        out_ref[0, pl.ds(c * CHUNK, CHUNK), :] = \
            acc_ref[pl.ds(c * CHUNK, CHUNK), :]

        for s in range(N_DEV - 1):
            send_idx = lax.rem(my + 1 + 2 * N_DEV - s, N_DEV)
            rdma = pltpu.make_async_remote_copy(
                src_ref=out_ref.at[0, pl.ds(send_idx * CHUNK, CHUNK), :],
                dst_ref=out_ref.at[0, pl.ds(send_idx * CHUNK, CHUNK), :],
                send_sem=ag_send_sems.at[s],
                recv_sem=ag_recv_sems.at[s],
                device_id=(right,),
                device_id_type=pl.DeviceIdType.MESH,
            )
            rdma.start()
            rdma.wait()

    return pl.pallas_call(
        body,
        out_shape=jax.ShapeDtypeStruct((1, SQ, D_MODEL), jnp.float32),
        in_specs=[
            pl.BlockSpec(memory_space=pltpu.MemorySpace.VMEM),
            pl.BlockSpec(memory_space=pltpu.MemorySpace.VMEM),
            pl.BlockSpec(memory_space=pl.ANY),
            pl.BlockSpec(memory_space=pl.ANY),
            pl.BlockSpec(memory_space=pltpu.MemorySpace.VMEM),
        ],
        out_specs=pl.BlockSpec(memory_space=pltpu.MemorySpace.VMEM),
        scratch_shapes=[
            pltpu.VMEM((SKV, H_PER, DH), jnp.float32),
            pltpu.VMEM((SKV, H_PER, DH), jnp.float32),
            pltpu.VMEM((SQ, D_MODEL), jnp.float32),
            pltpu.VMEM((N_DEV - 1, CHUNK, D_MODEL), jnp.float32),
            pltpu.SemaphoreType.DMA((2,)),
            pltpu.SemaphoreType.DMA((N_DEV - 1,)),
            pltpu.SemaphoreType.DMA((N_DEV - 1,)),
            pltpu.SemaphoreType.DMA((N_DEV - 1,)),
            pltpu.SemaphoreType.DMA((N_DEV - 1,)),
        ],
        compiler_params=pltpu.CompilerParams(collective_id=0),
    )(x, Wq, K_ext, V_ext, Wo)


# device time: 77350 ns/iter; 1.9247x vs baseline; 1.9247x over previous
import jax
import jax.numpy as jnp
from jax import lax
from jax.experimental import pallas as pl
from jax.experimental.pallas import tpu as pltpu

N_DEV = 8
SQ = 1024
SKV = 1024
D_MODEL = 1024
H_PER = 8
DH = 128
CHUNK = SQ // N_DEV
SCALE = 0.08838834764831843


def kernel(x, Wq, K_ext, V_ext, Wo):
    def body(x_ref, wq_ref, k_hbm, v_hbm, wo_ref, out_ref,
             k_vmem, v_vmem, acc_ref, acc_bf_ref, ag_send_buf,
             rs_recv, ag_recv,
             local_sems, rs_send_sems, rs_recv_sems,
             ag_send_sems, ag_recv_sems):
        my = lax.axis_index("i")

        barrier = pltpu.get_barrier_semaphore()
        for j_off in range(1, N_DEV):
            peer = lax.rem(my + j_off, N_DEV)
            pl.semaphore_signal(barrier, inc=1, device_id=(peer,),
                                device_id_type=pl.DeviceIdType.MESH)
        pl.semaphore_wait(barrier, N_DEV - 1)

        kcopy = pltpu.make_async_copy(
            k_hbm.at[0, :, pl.ds(my * H_PER, H_PER), :], k_vmem,
            local_sems.at[0])
        vcopy = pltpu.make_async_copy(
            v_hbm.at[0, :, pl.ds(my * H_PER, H_PER), :], v_vmem,
            local_sems.at[1])
        kcopy.start()
        vcopy.start()

        xb = x_ref[0].astype(jnp.bfloat16)
        wqb = wq_ref[...].astype(jnp.bfloat16)
        q = jnp.dot(xb, wqb, preferred_element_type=jnp.float32)

        qi = lax.broadcasted_iota(jnp.int32, (SQ, SKV), 0)
        ki = lax.broadcasted_iota(jnp.int32, (SQ, SKV), 1)
        mask = (jnp.abs(qi - ki) <= 128) | (ki < 32) | (qi < 32)

        kcopy.wait()
        vcopy.wait()

        acc = jnp.zeros((SQ, D_MODEL), jnp.float32)
        for h in range(H_PER):
            qh = q[:, h * DH:(h + 1) * DH].astype(jnp.bfloat16)
            kh = k_vmem[:, h, :].astype(jnp.bfloat16)
            s = lax.dot_general(qh, kh, (((1,), (1,)), ((), ())),
                                preferred_element_type=jnp.float32)
            s = jnp.where(mask, s * SCALE, -1e9)
            m = jnp.max(s, axis=1, keepdims=True)
            w = jnp.exp(s - m)
            w = w / jnp.sum(w, axis=1, keepdims=True)
            vh = v_vmem[:, h, :].astype(jnp.bfloat16)
            ctx = jnp.dot(w.astype(jnp.bfloat16), vh,
                          preferred_element_type=jnp.float32)
            wo_h = wo_ref[h * DH:(h + 1) * DH, :].astype(jnp.bfloat16)
            acc = acc + jnp.dot(ctx.astype(jnp.bfloat16), wo_h,
                                preferred_element_type=jnp.float32)
        acc_ref[...] = acc
        acc_bf_ref[...] = acc.astype(jnp.bfloat16)

        rs_rdmas = []
        for j_off in range(1, N_DEV):
            tgt = lax.rem(my + j_off, N_DEV)
            slot = (N_DEV - 1) - j_off
            rdma = pltpu.make_async_remote_copy(
                src_ref=acc_bf_ref.at[pl.ds(tgt * CHUNK, CHUNK), :],
                dst_ref=rs_recv.at[slot],
                send_sem=rs_send_sems.at[slot],
                recv_sem=rs_recv_sems.at[slot],
                device_id=(tgt,),
                device_id_type=pl.DeviceIdType.MESH,
            )
            rdma.start()
            rs_rdmas.append(rdma)

        total = acc_ref[pl.ds(my * CHUNK, CHUNK), :]
        for j_off in range(1, N_DEV):
            slot = (N_DEV - 1) - j_off
            rs_rdmas[j_off - 1].wait_recv()
            total = total + rs_recv[slot].astype(jnp.float32)

        out_ref[0, pl.ds(my * CHUNK, CHUNK), :] = total
        ag_send_buf[...] = total.astype(jnp.bfloat16)

        ag_rdmas = []
        for j_off in range(1, N_DEV):
            tgt = lax.rem(my + j_off, N_DEV)
            slot = (N_DEV - 1) - j_off
            rdma = pltpu.make_async_remote_copy(
                src_ref=ag_send_buf,
                dst_ref=ag_recv.at[slot],
                send_sem=ag_send_sems.at[slot],
                recv_sem=ag_recv_sems.at[slot],
                device_id=(tgt,),
                device_id_type=pl.DeviceIdType.MESH,
            )
            rdma.start()
            ag_rdmas.append(rdma)

        for k_off in range(1, N_DEV):
            slot = (N_DEV - 1) - k_off
            ag_rdmas[k_off - 1].wait_recv()
            src_dev = lax.rem(my + N_DEV - k_off, N_DEV)
            out_ref[0, pl.ds(src_dev * CHUNK, CHUNK), :] = \
                ag_recv[slot].astype(jnp.float32)

        for rdma in rs_rdmas + ag_rdmas:
            rdma.wait_send()

    return pl.pallas_call(
        body,
        out_shape=jax.ShapeDtypeStruct((1, SQ, D_MODEL), jnp.float32),
        in_specs=[
            pl.BlockSpec(memory_space=pltpu.MemorySpace.VMEM),
            pl.BlockSpec(memory_space=pltpu.MemorySpace.VMEM),
            pl.BlockSpec(memory_space=pl.ANY),
            pl.BlockSpec(memory_space=pl.ANY),
            pl.BlockSpec(memory_space=pltpu.MemorySpace.VMEM),
        ],
        out_specs=pl.BlockSpec(memory_space=pltpu.MemorySpace.VMEM),
        scratch_shapes=[
            pltpu.VMEM((SKV, H_PER, DH), jnp.float32),
            pltpu.VMEM((SKV, H_PER, DH), jnp.float32),
            pltpu.VMEM((SQ, D_MODEL), jnp.float32),
            pltpu.VMEM((SQ, D_MODEL), jnp.bfloat16),
            pltpu.VMEM((CHUNK, D_MODEL), jnp.bfloat16),
            pltpu.VMEM((N_DEV - 1, CHUNK, D_MODEL), jnp.bfloat16),
            pltpu.VMEM((N_DEV - 1, CHUNK, D_MODEL), jnp.bfloat16),
            pltpu.SemaphoreType.DMA((2,)),
            pltpu.SemaphoreType.DMA((N_DEV - 1,)),
            pltpu.SemaphoreType.DMA((N_DEV - 1,)),
            pltpu.SemaphoreType.DMA((N_DEV - 1,)),
            pltpu.SemaphoreType.DMA((N_DEV - 1,)),
        ],
        compiler_params=pltpu.CompilerParams(collective_id=0),
    )(x, Wq, K_ext, V_ext, Wo)
